# baseline (speedup 1.0000x reference)
import functools

import jax
import jax.numpy as jnp
from jax import lax
from jax.experimental import pallas as pl
from jax.experimental.pallas import tpu as pltpu
from jax.experimental.pallas import tpu_sc as plsc

_N = 50000
_E = 800000

_NC = 2
_NS = 16
_NW = _NC * _NS
_CH = 128
_K = 196
_EPW = _K * _CH
_EPAD = _NW * _EPW
_CW = 256
_G = 7
_KS = 196
_K0 = 112
_KI = 98
_RPS = 3128
_NACC = _NS * _RPS
_NBUF = 3
_LOOK = 2
_F = 32
_W = 128

_mesh = plsc.VectorSubcoreMesh(
    core_axis_name="c", subcore_axis_name="s", num_cores=_NC, num_subcores=_NS)


def _make_prop(T):
    @functools.partial(
        pl.kernel,
        out_type=jax.ShapeDtypeStruct((_NC, _NACC, _W), jnp.float32),
        mesh=_mesh,
        scratch_types=[
            pltpu.VMEM((_G, _CW), jnp.int32),
            pltpu.VMEM((_G, _CW), jnp.int32),
            tuple(pltpu.VMEM((_CW, _F), jnp.float32) for _ in range(_NBUF)),
            pltpu.SemaphoreType.DMA((_NBUF,)),
            pltpu.SemaphoreType.DMA((_NBUF,)),
            pltpu.VMEM_SHARED((_NACC, _F), jnp.float32),
        ],
        compiler_params=pltpu.CompilerParams(use_tc_tiling_on_sc=False),
    )
    def prop(g_hbm, src_hbm, dst_hbm, zeros_hbm, out_hbm,
             src_v, dst_v, bufs, gsem, ssem, acc):
        c = lax.axis_index("c")
        s = lax.axis_index("s")
        base = s * _KS + c * _K0
        ngroups = 16 - 4 * c
        r0 = s * _RPS
        pltpu.sync_copy(zeros_hbm.at[pl.ds(r0, _RPS)],
                        acc.at[pl.ds(r0, _RPS)])
        plsc.subcore_barrier()

        @pl.loop(0, T)
        def _passes(t):
            t32 = t * _F

            @pl.loop(0, ngroups)
            def _groups(g):
                g0 = base + g * _G
                pltpu.sync_copy(src_hbm.at[t, pl.ds(g0, _G)], src_v)
                pltpu.sync_copy(dst_hbm.at[pl.ds(g0, _G)], dst_v)
                for j in range(_LOOK):
                    pltpu.async_copy(g_hbm.at[src_v.at[j]],
                                     bufs[j % _NBUF], gsem.at[j % _NBUF])
                for j in range(_G):
                    b = j % _NBUF
                    pltpu.make_async_copy(g_hbm.at[src_v.at[j]],
                                          bufs[b], gsem.at[b]).wait()
                    pltpu.make_async_copy(bufs[b], acc.at[dst_v.at[j]],
                                          ssem.at[b]).start(add=True)
                    jn = j + _LOOK
                    if jn < _G:
                        bn = jn % _NBUF
                        if jn >= _NBUF:
                            pltpu.make_async_copy(
                                bufs[bn], acc.at[dst_v.at[jn]],
                                ssem.at[bn]).wait()
                        pltpu.async_copy(g_hbm.at[src_v.at[jn]],
                                         bufs[bn], gsem.at[bn])
                for j in range(_G - _NBUF, _G):
                    b = j % _NBUF
                    pltpu.make_async_copy(bufs[b], acc.at[dst_v.at[j]],
                                          ssem.at[b]).wait()

            plsc.subcore_barrier()
            pltpu.sync_copy(acc.at[pl.ds(r0, _RPS)],
                            out_hbm.at[c, pl.ds(r0, _RPS), pl.ds(t32, _F)])
            pltpu.sync_copy(zeros_hbm.at[pl.ds(r0, _RPS)],
                            acc.at[pl.ds(r0, _RPS)])
            plsc.subcore_barrier()

    return prop


_prop1 = _make_prop(1)
_prop4 = _make_prop(4)


@functools.partial(
    pl.kernel,
    out_type=jax.ShapeDtypeStruct((_NC, _NACC, _W), jnp.float32),
    mesh=_mesh,
    scratch_types=[
        pltpu.VMEM((_G, _CW), jnp.int32),
        pltpu.VMEM((_CW, 8), jnp.float32),
        pltpu.SemaphoreType.DMA,
        pltpu.VMEM_SHARED((_NACC, 8), jnp.float32),
    ],
    compiler_params=pltpu.CompilerParams(use_tc_tiling_on_sc=False),
)
def _indeg(dst_hbm, zeros_hbm, ones_hbm, out_hbm, dst_v, ones_v, sem, acc):
    c = lax.axis_index("c")
    s = lax.axis_index("s")
    base = s * _KS + c * _KI
    pltpu.sync_copy(ones_hbm, ones_v)
    r0 = s * _RPS
    pltpu.sync_copy(zeros_hbm.at[pl.ds(r0, _RPS)], acc.at[pl.ds(r0, _RPS)])
    plsc.subcore_barrier()

    @pl.loop(0, 14)
    def _groups(g):
        pltpu.sync_copy(dst_hbm.at[pl.ds(base + g * _G, _G)], dst_v)
        for j in range(_G):
            pltpu.make_async_copy(ones_v, acc.at[dst_v.at[j]],
                                  sem).start(add=True)
        for j in range(_G):
            pltpu.make_async_copy(ones_v, acc.at[dst_v.at[j]], sem).wait()

    plsc.subcore_barrier()
    pltpu.sync_copy(acc.at[pl.ds(r0, _RPS)],
                    out_hbm.at[c, pl.ds(r0, _RPS), pl.ds(0, 8)])


_BN = 3128
_GRID = _NACC // _BN


def _row_spec(f):
    return pl.BlockSpec((_BN, f), lambda i: (i, 0))


def _part_specs():
    return [pl.BlockSpec((1, _BN, _W), lambda i: (0, i, 0)),
            pl.BlockSpec((1, _BN, _W), lambda i: (1, i, 0))]


def _full_spec(shape):
    nd = len(shape)
    return pl.BlockSpec(shape, lambda i: (0,) * nd)


def _padw(v):
    return jnp.concatenate(
        [v, jnp.zeros((v.shape[0], _W - v.shape[1]), jnp.float32)], axis=1)


def _tc0_body(xp, w1, h0):
    h0[...] = jnp.dot(xp[...], w1[...], preferred_element_type=jnp.float32)


def _tc1_body(h0, p0, p1, g1, dinv, rc):
    cnt0 = p0[0][:, 0:1] + p1[0][:, 0:1]
    dinv_v = lax.rsqrt(cnt0 + 1.0)
    g1[...] = _padw(dinv_v * h0[...])
    dinv[...] = dinv_v
    rc[...] = 1.0 / jnp.maximum(cnt0, 1.0)


def _tc2_body(q0, q1, g1, dinv, b1, h1):
    ssum = q0[0][:, 0:_F] + q1[0][:, 0:_F] + g1[:, 0:_F]
    h1[...] = _padw(jnp.maximum(dinv[...] * ssum + b1[...], 0.0))


def _tc3_body(r0, r1, rc, h1, w2l, b2, w2r, h2):
    agg = (r0[0][:, 0:_F] + r1[0][:, 0:_F]) * rc[...]
    h2[...] = jnp.maximum(
        jnp.dot(agg, w2l[...], preferred_element_type=jnp.float32) + b2[...]
        + jnp.dot(h1[:, 0:_F], w2r[...],
                  preferred_element_type=jnp.float32), 0.0)


def _tc4_body(s0, s1, rc, h2, w3l, b3, w3r, w4l, h3, g4):
    agg = (s0[0] + s1[0]) * rc[...]
    h3_v = jnp.maximum(
        jnp.dot(agg, w3l[...], preferred_element_type=jnp.float32) + b3[...]
        + jnp.dot(h2[...], w3r[...], preferred_element_type=jnp.float32), 0.0)
    h3[...] = h3_v
    g4[...] = _padw(jnp.dot(h3_v, w4l[...],
                            preferred_element_type=jnp.float32))


def _tc5_body(t0, t1, rc, h3, w4r, b4, wm1, bm1, wm2, bm2, out):
    h4 = ((t0[0][:, 0:_F] + t1[0][:, 0:_F]) * rc[...] + b4[...]
          + jnp.dot(h3[...], w4r[...], preferred_element_type=jnp.float32))
    h5 = jnp.maximum(
        jnp.dot(h4, wm1[...], preferred_element_type=jnp.float32) + bm1[...],
        0.0)
    z = jnp.dot(h5, wm2[...], preferred_element_type=jnp.float32) + bm2[...]
    out[...] = 1.0 / (1.0 + jnp.exp(-z))


def _sds(shape):
    return jax.ShapeDtypeStruct(shape, jnp.float32)


def kernel(x, edge_index, W1, b1, W2l, b2, W2r, W3l, b3, W3r,
           W4l, b4, W4r, Wm1, bm1, Wm2, bm2):
    src = edge_index[0]
    dst = edge_index[1]
    pad = _EPAD - _E
    srcp = jnp.concatenate(
        [src, jnp.zeros((pad,), jnp.int32)]).reshape(_NW * _KS // 2, _CW)
    dstp = jnp.concatenate(
        [dst, jnp.full((pad,), _N, jnp.int32)]).reshape(_NW * _KS // 2, _CW)
    srcp1 = (srcp * 4)[None]
    srcp4 = srcp1 + jnp.arange(4, dtype=jnp.int32)[:, None, None]
    zeros32 = jnp.zeros((_NACC, _F), jnp.float32)
    zeros8 = jnp.zeros((_NACC, 8), jnp.float32)
    ones8 = jnp.ones((_CW, 8), jnp.float32)

    pc = _indeg(dstp, zeros8, ones8)

    xp = jnp.pad(x, ((0, _NACC - _N), (0, 5)))
    W1p = jnp.pad(W1, ((0, 5), (0, 0)))

    h0 = pl.pallas_call(
        _tc0_body,
        grid=(_GRID,),
        in_specs=[_row_spec(8), _full_spec((8, 32))],
        out_specs=_row_spec(32),
        out_shape=_sds((_NACC, 32)),
    )(xp, W1p)

    g1, dinv, rc = pl.pallas_call(
        _tc1_body,
        grid=(_GRID,),
        in_specs=[_row_spec(32)] + _part_specs(),
        out_specs=[_row_spec(_W), _row_spec(1), _row_spec(1)],
        out_shape=[_sds((_NACC, _W)), _sds((_NACC, 1)), _sds((_NACC, 1))],
    )(h0, pc, pc)

    q = _prop1(g1.reshape(4 * _NACC, _F), srcp1, dstp, zeros32)
    h1 = pl.pallas_call(
        _tc2_body,
        grid=(_GRID,),
        in_specs=_part_specs() + [_row_spec(_W), _row_spec(1),
                                  _full_spec((32,))],
        out_specs=_row_spec(_W),
        out_shape=_sds((_NACC, _W)),
    )(q, q, g1, dinv, b1)

    r = _prop1(h1.reshape(4 * _NACC, _F), srcp1, dstp, zeros32)
    h2 = pl.pallas_call(
        _tc3_body,
        grid=(_GRID,),
        in_specs=_part_specs() + [_row_spec(1), _row_spec(_W),
                                  _full_spec((32, 128)), _full_spec((128,)),
                                  _full_spec((32, 128))],
        out_specs=_row_spec(_W),
        out_shape=_sds((_NACC, _W)),
    )(r, r, rc, h1, W2l, b2, W2r)

    sall = _prop4(h2.reshape(4 * _NACC, _F), srcp4, dstp, zeros32)

    h3, g4 = pl.pallas_call(
        _tc4_body,
        grid=(_GRID,),
        in_specs=_part_specs() + [_row_spec(1), _row_spec(_W),
                                  _full_spec((128, 128)), _full_spec((128,)),
                                  _full_spec((128, 128)),
                                  _full_spec((128, 32))],
        out_specs=[_row_spec(_W), _row_spec(_W)],
        out_shape=[_sds((_NACC, _W)), _sds((_NACC, _W))],
    )(sall, sall, rc, h2, W3l, b3, W3r, W4l)

    t = _prop1(g4.reshape(4 * _NACC, _F), srcp1, dstp, zeros32)
    o = pl.pallas_call(
        _tc5_body,
        grid=(_GRID,),
        in_specs=_part_specs() + [_row_spec(1), _row_spec(_W),
                                  _full_spec((128, 32)), _full_spec((32,)),
                                  _full_spec((32, 16)), _full_spec((16,)),
                                  _full_spec((16, 1)), _full_spec((1,))],
        out_specs=_row_spec(1),
        out_shape=_sds((_NACC, 1)),
    )(t, t, rc, h3, W4r, b4, Wm1, bm1, Wm2, bm2)

    return o[:_N, 0]


# speedup vs baseline: 1.1023x; 1.1023x over previous
import functools

import jax
import jax.numpy as jnp
from jax import lax
from jax.experimental import pallas as pl
from jax.experimental.pallas import tpu as pltpu
from jax.experimental.pallas import tpu_sc as plsc

_N = 50000
_E = 800000

_NC = 2
_NS = 16
_NW = _NC * _NS
_CH = 128
_K = 196
_EPW = _K * _CH
_EPAD = _NW * _EPW
_G = 28
_NG = _K // _G
_KS = 2 * _K
_K0 = 224
_RPS = 3128
_NACC = _NS * _RPS
_NBUF = 5
_LOOK = 4
_F = 32
_W = 128

_mesh = plsc.VectorSubcoreMesh(
    core_axis_name="c", subcore_axis_name="s", num_cores=_NC, num_subcores=_NS)


def _make_prop(T):
    @functools.partial(
        pl.kernel,
        out_type=jax.ShapeDtypeStruct((_NC, _NACC, _W), jnp.float32),
        mesh=_mesh,
        scratch_types=[
            pltpu.VMEM((_G, _CH), jnp.int32),
            pltpu.VMEM((_G, _CH), jnp.int32),
            tuple(pltpu.VMEM((_CH, _F), jnp.float32) for _ in range(_NBUF)),
            pltpu.SemaphoreType.DMA((_NBUF,)),
            pltpu.SemaphoreType.DMA((_NBUF,)),
            pltpu.VMEM_SHARED((_NACC, _F), jnp.float32),
        ],
        compiler_params=pltpu.CompilerParams(use_tc_tiling_on_sc=False),
    )
    def prop(g_hbm, src_hbm, dst_hbm, zeros_hbm, out_hbm,
             src_v, dst_v, bufs, gsem, ssem, acc):
        c = lax.axis_index("c")
        s = lax.axis_index("s")
        base = s * _KS + c * _K0
        ngroups = 8 - 2 * c
        r0 = s * _RPS
        pltpu.sync_copy(zeros_hbm.at[pl.ds(r0, _RPS)],
                        acc.at[pl.ds(r0, _RPS)])
        plsc.subcore_barrier()

        @pl.loop(0, T)
        def _passes(t):
            t32 = t * _F

            @pl.loop(0, ngroups)
            def _groups(g):
                g0 = base + g * _G
                pltpu.sync_copy(src_hbm.at[t, pl.ds(g0, _G)], src_v)
                pltpu.sync_copy(dst_hbm.at[pl.ds(g0, _G)], dst_v)
                for j in range(_LOOK):
                    pltpu.async_copy(
                        g_hbm.at[src_v.at[j]],
                        bufs[j % _NBUF], gsem.at[j % _NBUF])
                for j in range(_G):
                    b = j % _NBUF
                    pltpu.make_async_copy(
                        g_hbm.at[src_v.at[j]],
                        bufs[b], gsem.at[b]).wait()
                    pltpu.make_async_copy(bufs[b], acc.at[dst_v.at[j]],
                                          ssem.at[b]).start(add=True)
                    jn = j + _LOOK
                    if jn < _G:
                        bn = jn % _NBUF
                        if jn >= _NBUF:
                            pltpu.make_async_copy(
                                bufs[bn], acc.at[dst_v.at[jn]],
                                ssem.at[bn]).wait()
                        pltpu.async_copy(
                            g_hbm.at[src_v.at[jn]],
                            bufs[bn], gsem.at[bn])
                for j in range(_G - _NBUF, _G):
                    b = j % _NBUF
                    pltpu.make_async_copy(bufs[b], acc.at[dst_v.at[j]],
                                          ssem.at[b]).wait()

            plsc.subcore_barrier()
            pltpu.sync_copy(acc.at[pl.ds(r0, _RPS)],
                            out_hbm.at[c, pl.ds(r0, _RPS), pl.ds(t32, _F)])
            pltpu.sync_copy(zeros_hbm.at[pl.ds(r0, _RPS)],
                            acc.at[pl.ds(r0, _RPS)])
            plsc.subcore_barrier()

    return prop


_prop1 = _make_prop(1)
_prop4 = _make_prop(4)


@functools.partial(
    pl.kernel,
    out_type=jax.ShapeDtypeStruct((_NC, _NACC, _W), jnp.float32),
    mesh=_mesh,
    scratch_types=[
        pltpu.VMEM((_G, _CH), jnp.int32),
        pltpu.VMEM((_CH, 8), jnp.float32),
        pltpu.SemaphoreType.DMA,
        pltpu.VMEM_SHARED((_NACC, 8), jnp.float32),
    ],
    compiler_params=pltpu.CompilerParams(use_tc_tiling_on_sc=False),
)
def _indeg(dst_hbm, zeros_hbm, ones_hbm, out_hbm, dst_v, ones_v, sem, acc):
    c = lax.axis_index("c")
    s = lax.axis_index("s")
    base = s * _KS + c * _K
    pltpu.sync_copy(ones_hbm, ones_v)
    r0 = s * _RPS
    pltpu.sync_copy(zeros_hbm.at[pl.ds(r0, _RPS)], acc.at[pl.ds(r0, _RPS)])
    plsc.subcore_barrier()

    @pl.loop(0, _NG)
    def _groups(g):
        pltpu.sync_copy(dst_hbm.at[pl.ds(base + g * _G, _G)], dst_v)
        for j in range(_G):
            pltpu.make_async_copy(ones_v, acc.at[dst_v.at[j]],
                                  sem).start(add=True)
        for j in range(_G):
            pltpu.make_async_copy(ones_v, acc.at[dst_v.at[j]], sem).wait()

    plsc.subcore_barrier()
    pltpu.sync_copy(acc.at[pl.ds(r0, _RPS)],
                    out_hbm.at[c, pl.ds(r0, _RPS), pl.ds(0, 8)])


_BN = 3128
_GRID = _NACC // _BN


def _row_spec(f):
    return pl.BlockSpec((_BN, f), lambda i: (i, 0))


def _part_specs():
    return [pl.BlockSpec((1, _BN, _W), lambda i: (0, i, 0)),
            pl.BlockSpec((1, _BN, _W), lambda i: (1, i, 0))]


def _full_spec(shape):
    nd = len(shape)
    return pl.BlockSpec(shape, lambda i: (0,) * nd)


def _padw(v):
    return jnp.concatenate(
        [v, jnp.zeros((v.shape[0], _W - v.shape[1]), jnp.float32)], axis=1)


def _tc0_body(xp, w1, h0):
    h0[...] = jnp.dot(xp[...], w1[...], preferred_element_type=jnp.float32)


def _tc1_body(h0, p0, p1, g1, dinv, rc):
    cnt0 = p0[0][:, 0:1] + p1[0][:, 0:1]
    dinv_v = lax.rsqrt(cnt0 + 1.0)
    g1[...] = _padw(dinv_v * h0[...])
    dinv[...] = dinv_v
    rc[...] = 1.0 / jnp.maximum(cnt0, 1.0)


def _tc2_body(q0, q1, g1, dinv, b1, h1):
    ssum = q0[0][:, 0:_F] + q1[0][:, 0:_F] + g1[:, 0:_F]
    h1[...] = _padw(jnp.maximum(dinv[...] * ssum + b1[...], 0.0))


def _tc3_body(r0, r1, rc, h1, w2l, b2, w2r, h2):
    agg = (r0[0][:, 0:_F] + r1[0][:, 0:_F]) * rc[...]
    h2[...] = jnp.maximum(
        jnp.dot(agg, w2l[...], preferred_element_type=jnp.float32) + b2[...]
        + jnp.dot(h1[:, 0:_F], w2r[...],
                  preferred_element_type=jnp.float32), 0.0)


def _tc4_body(s0, s1, rc, h2, w3l, b3, w3r, w4l, h3, g4):
    agg = (s0[0] + s1[0]) * rc[...]
    h3_v = jnp.maximum(
        jnp.dot(agg, w3l[...], preferred_element_type=jnp.float32) + b3[...]
        + jnp.dot(h2[...], w3r[...], preferred_element_type=jnp.float32), 0.0)
    h3[...] = h3_v
    g4[...] = _padw(jnp.dot(h3_v, w4l[...],
                            preferred_element_type=jnp.float32))


def _tc5_body(t0, t1, rc, h3, w4r, b4, wm1, bm1, wm2, bm2, out):
    h4 = ((t0[0][:, 0:_F] + t1[0][:, 0:_F]) * rc[...] + b4[...]
          + jnp.dot(h3[...], w4r[...], preferred_element_type=jnp.float32))
    h5 = jnp.maximum(
        jnp.dot(h4, wm1[...], preferred_element_type=jnp.float32) + bm1[...],
        0.0)
    z = jnp.dot(h5, wm2[...], preferred_element_type=jnp.float32) + bm2[...]
    out[...] = 1.0 / (1.0 + jnp.exp(-z))


def _sds(shape):
    return jax.ShapeDtypeStruct(shape, jnp.float32)


def kernel(x, edge_index, W1, b1, W2l, b2, W2r, W3l, b3, W3r,
           W4l, b4, W4r, Wm1, bm1, Wm2, bm2):
    src = edge_index[0]
    dst = edge_index[1]
    pad = _EPAD - _E
    srcp = jnp.concatenate(
        [src, jnp.zeros((pad,), jnp.int32)]).reshape(_NW * _K, _CH)
    dstp = jnp.concatenate(
        [dst, jnp.full((pad,), _N, jnp.int32)]).reshape(_NW * _K, _CH)
    srcp1 = (srcp * 4)[None]
    srcp4 = srcp1 + jnp.arange(4, dtype=jnp.int32)[:, None, None]
    zeros32 = jnp.zeros((_NACC, _F), jnp.float32)
    zeros8 = jnp.zeros((_NACC, 8), jnp.float32)
    ones8 = jnp.ones((_CH, 8), jnp.float32)

    pc = _indeg(dstp, zeros8, ones8)

    xp = jnp.pad(x, ((0, _NACC - _N), (0, 5)))
    W1p = jnp.pad(W1, ((0, 5), (0, 0)))

    h0 = pl.pallas_call(
        _tc0_body,
        grid=(_GRID,),
        in_specs=[_row_spec(8), _full_spec((8, 32))],
        out_specs=_row_spec(32),
        out_shape=_sds((_NACC, 32)),
    )(xp, W1p)

    g1, dinv, rc = pl.pallas_call(
        _tc1_body,
        grid=(_GRID,),
        in_specs=[_row_spec(32)] + _part_specs(),
        out_specs=[_row_spec(_W), _row_spec(1), _row_spec(1)],
        out_shape=[_sds((_NACC, _W)), _sds((_NACC, 1)), _sds((_NACC, 1))],
    )(h0, pc, pc)

    q = _prop1(g1.reshape(4 * _NACC, _F), srcp1, dstp, zeros32)
    h1 = pl.pallas_call(
        _tc2_body,
        grid=(_GRID,),
        in_specs=_part_specs() + [_row_spec(_W), _row_spec(1),
                                  _full_spec((32,))],
        out_specs=_row_spec(_W),
        out_shape=_sds((_NACC, _W)),
    )(q, q, g1, dinv, b1)

    r = _prop1(h1.reshape(4 * _NACC, _F), srcp1, dstp, zeros32)
    h2 = pl.pallas_call(
        _tc3_body,
        grid=(_GRID,),
        in_specs=_part_specs() + [_row_spec(1), _row_spec(_W),
                                  _full_spec((32, 128)), _full_spec((128,)),
                                  _full_spec((32, 128))],
        out_specs=_row_spec(_W),
        out_shape=_sds((_NACC, _W)),
    )(r, r, rc, h1, W2l, b2, W2r)

    sall = _prop4(h2.reshape(4 * _NACC, _F), srcp4, dstp, zeros32)

    h3, g4 = pl.pallas_call(
        _tc4_body,
        grid=(_GRID,),
        in_specs=_part_specs() + [_row_spec(1), _row_spec(_W),
                                  _full_spec((128, 128)), _full_spec((128,)),
                                  _full_spec((128, 128)),
                                  _full_spec((128, 32))],
        out_specs=[_row_spec(_W), _row_spec(_W)],
        out_shape=[_sds((_NACC, _W)), _sds((_NACC, _W))],
    )(sall, sall, rc, h2, W3l, b3, W3r, W4l)

    t = _prop1(g4.reshape(4 * _NACC, _F), srcp1, dstp, zeros32)
    o = pl.pallas_call(
        _tc5_body,
        grid=(_GRID,),
        in_specs=_part_specs() + [_row_spec(1), _row_spec(_W),
                                  _full_spec((128, 32)), _full_spec((32,)),
                                  _full_spec((32, 16)), _full_spec((16,)),
                                  _full_spec((16, 1)), _full_spec((1,))],
        out_specs=_row_spec(1),
        out_shape=_sds((_NACC, 1)),
    )(t, t, rc, h3, W4r, b4, Wm1, bm1, Wm2, bm2)

    return o[:_N, 0]
